# Initial kernel scaffold; baseline (speedup 1.0000x reference)
import jax
import jax.numpy as jnp
from jax.experimental import pallas as pl


def kernel(x, edge_index, params):
    raise NotImplementedError("write your pallas kernel here")


# speedup vs baseline: 5.0470x; 5.0470x over previous
import functools

import jax
import jax.numpy as jnp
from jax import lax
from jax.experimental import pallas as pl
from jax.experimental.pallas import tpu as pltpu
from jax.experimental.pallas import tpu_sc as plsc


NC = 2
NS = 16
K = 80


@functools.partial(jax.jit, static_argnums=(4, 5, 6))
def _sc_partial_agg(x, src3, dst3, zeros_nd, n, d, ch):
    rpt = (n // NS) // 8 * 8
    rpt_last = n - (NS - 1) * rpt
    mesh = plsc.VectorSubcoreMesh(core_axis_name="c", subcore_axis_name="s")

    @functools.partial(
        pl.kernel,
        out_type=jax.ShapeDtypeStruct((NC, n, d), jnp.float32),
        mesh=mesh,
        scratch_types=[
            pltpu.VMEM((ch, K), jnp.int32),
            pltpu.VMEM((ch, K), jnp.int32),
            pltpu.VMEM((K, d), jnp.float32),
            pltpu.VMEM_SHARED((n, d), jnp.float32),
            pltpu.SemaphoreType.DMA,
        ],
    )
    def agg_kernel(x_hbm, src_hbm, dst_hbm, z_hbm, out_hbm, src_v, dst_v,
                   rows_v, acc_sh, sem):
        c = lax.axis_index("c")
        s = lax.axis_index("s")
        wid = c * NS + s

        r0 = s * rpt

        @pl.when(s < NS - 1)
        def _():
            pltpu.sync_copy(z_hbm.at[pl.ds(r0, rpt)],
                            acc_sh.at[pl.ds(r0, rpt)])

        @pl.when(s == NS - 1)
        def _():
            pltpu.sync_copy(z_hbm.at[pl.ds((NS - 1) * rpt, rpt_last)],
                            acc_sh.at[pl.ds((NS - 1) * rpt, rpt_last)])

        pltpu.sync_copy(src_hbm.at[wid], src_v)
        pltpu.sync_copy(dst_hbm.at[wid], dst_v)
        plsc.subcore_barrier()

        def chunk_body(j, carry):
            pltpu.async_copy(x_hbm.at[src_v.at[j]], rows_v, sem).wait()
            pltpu.sync_copy(rows_v, acc_sh.at[dst_v.at[j]], add=True)
            return carry

        lax.fori_loop(0, ch, chunk_body, 0, unroll=False)
        plsc.subcore_barrier()

        @pl.when(s < NS - 1)
        def _():
            pltpu.sync_copy(acc_sh.at[pl.ds(r0, rpt)],
                            out_hbm.at[c, pl.ds(r0, rpt)])

        @pl.when(s == NS - 1)
        def _():
            pltpu.sync_copy(acc_sh.at[pl.ds((NS - 1) * rpt, rpt_last)],
                            out_hbm.at[c, pl.ds((NS - 1) * rpt, rpt_last)])

    return agg_kernel(x, src3, dst3, zeros_nd)


def _tree442(acc):
    a = acc[0:4] + acc[4:8]
    b = a[0:2] + a[2:4]
    return b[0:1] + b[1:2]


def _mlp_body(p0_ref, p1_ref, x_ref, w1_ref, b1_ref, g_ref, beta_ref,
              w2_ref, b2_ref, out_ref, m_ref, sq_ref):
    n = m_ref.shape[0]
    h = x_ref[...] + (p0_ref[...] + p1_ref[...])
    m_ref[...] = jnp.dot(h, w1_ref[...],
                         preferred_element_type=jnp.float32) + b1_ref[...]
    m = m_ref[...]
    s = jnp.sum(m, axis=0, keepdims=True)
    mean = s * jnp.float32(1.0 / n)
    dmu = m - mean
    sq_ref[...] = dmu * dmu

    def _accrange(lo, hi):
        def red(i, acc):
            return acc + sq_ref[pl.ds(i * 8, 8), :]
        return lax.fori_loop(lo, hi, red, jnp.zeros((8, 128), jnp.float32))

    half = (n // 8) // 2
    vs = _tree442(_accrange(0, half)) + _tree442(_accrange(half, n // 8))
    var = vs * jnp.float32(1.0 / n)
    denom = jnp.sqrt(var + jnp.float32(1e-5))
    r = jnp.maximum((m - mean) / denom * g_ref[...] + beta_ref[...],
                    jnp.float32(0))
    o = jnp.dot(r, w2_ref[...], preferred_element_type=jnp.float32)
    out_ref[...] = jnp.maximum(o + b2_ref[...], jnp.float32(0))


@jax.jit
def _tc_mlp(p0, p1, x, w1, b1, g, beta, w2, b2):
    n, d = x.shape
    return pl.pallas_call(
        _mlp_body,
        out_shape=jax.ShapeDtypeStruct((n, d), jnp.float32),
        scratch_shapes=[pltpu.VMEM((n, d), jnp.float32),
                        pltpu.VMEM((n, d), jnp.float32)],
        compiler_params=pltpu.CompilerParams(vmem_limit_bytes=134217728),
    )(p0, p1, x, w1, b1.reshape(1, -1), g.reshape(1, -1),
      beta.reshape(1, -1), w2, b2.reshape(1, -1))


def kernel(x, edge_index, params):
    n, d = x.shape
    e = edge_index.shape[1]
    nw = NC * NS
    assert e % (nw * K) == 0
    ch = e // (nw * K)

    perm = jnp.argsort(edge_index[1], stable=True)
    src3 = edge_index[0][perm].reshape(nw, ch, K)
    dst3 = edge_index[1][perm].reshape(nw, ch, K)
    zeros_nd = jnp.zeros((n, d), jnp.float32)

    h = x
    for p in params:
        parts = _sc_partial_agg(h, src3, dst3, zeros_nd, n, d, ch)
        w2 = p["W2"]
        b2 = p["b2"]
        dout = w2.shape[1]
        if dout < d:
            w2 = jnp.pad(w2, ((0, 0), (0, d - dout)))
            b2 = jnp.pad(b2, ((0, d - dout),))
        h = _tc_mlp(parts[0], parts[1], h, p["W1"], p["b1"], p["g"],
                    p["beta"], w2, b2)
    dout_last = params[-1]["W2"].shape[1]
    if dout_last < d:
        h = h[:, :dout_last]
    return h
